# Initial kernel scaffold; baseline (speedup 1.0000x reference)
import jax
import jax.numpy as jnp
from jax.experimental import pallas as pl


def kernel(tenInput, g):
    raise NotImplementedError("write your pallas kernel here")


# speedup vs baseline: 1.2058x; 1.2058x over previous
import functools

import jax
import jax.numpy as jnp
from jax import lax
from jax.experimental import pallas as pl
from jax.experimental.pallas import tpu as pltpu
from jax.experimental.pallas import tpu_sc as plsc

N, C, H, W = 4, 96, 512, 512
HW = H * W
P_TOTAL = N * HW

NC, NS, L = 2, 16, 16
NW = NC * NS
PW = P_TOTAL // NW
CH = 128
N_CHUNKS = PW // CH


def _sampler_body(tab, gx, gy, out, gxv, gyv, idxv, wflat, rows, obuf, sem):
    wid = lax.axis_index("s") * NC + lax.axis_index("c")
    base = wid * PW
    row_base = (wid // (NW // N)) * HW

    def chunk_body(i, _):
        p0 = base + i * CH
        pltpu.sync_copy(gx.at[pl.ds(p0, CH)], gxv)
        pltpu.sync_copy(gy.at[pl.ds(p0, CH)], gyv)

        for v in range(CH // L):
            sl = pl.ds(v * L, L)
            x = (gxv[sl] + 1.0) * ((W - 1) / 2.0)
            y = (gyv[sl] + 1.0) * ((H - 1) / 2.0)
            xi = jnp.minimum(jnp.maximum(x.astype(jnp.int32), 0), W - 2)
            yi = jnp.minimum(jnp.maximum(y.astype(jnp.int32), 0), H - 2)
            fx = x - xi.astype(jnp.float32)
            fy = y - yi.astype(jnp.float32)
            ra = yi * W + xi + row_base
            idxv[0, sl] = ra
            idxv[1, sl] = ra + W
            idxv[2, sl] = ra + 1
            idxv[3, sl] = ra + (W + 1)
            ex = 1.0 - fx
            ey = 1.0 - fy
            wflat[pl.ds(v * L, L)] = ex * ey
            wflat[pl.ds(CH + v * L, L)] = ex * fy
            wflat[pl.ds(2 * CH + v * L, L)] = fx * ey
            wflat[pl.ds(3 * CH + v * L, L)] = fx * fy

        cps = [pltpu.async_copy(tab.at[idxv.at[j]], rows.at[j], sem)
               for j in range(4)]
        for cp in cps:
            cp.wait()

        def pos_body(p, _):
            pv = jnp.full((L,), p, dtype=jnp.int32)
            wa = plsc.load_gather(wflat, [pv])
            wb = plsc.load_gather(wflat, [pv + CH])
            wc = plsc.load_gather(wflat, [pv + 2 * CH])
            wd = plsc.load_gather(wflat, [pv + 3 * CH])
            for j in range(C // L):
                s2 = pl.ds(j * L, L)
                obuf[p, s2] = (rows[0, p, s2] * wa + rows[1, p, s2] * wb +
                               rows[2, p, s2] * wc + rows[3, p, s2] * wd)
            return 0

        lax.fori_loop(0, CH, pos_body, 0, unroll=False)
        pltpu.sync_copy(obuf, out.at[pl.ds(p0, CH)])
        return 0

    lax.fori_loop(0, N_CHUNKS, chunk_body, 0, unroll=False)


_sampler = pl.kernel(
    _sampler_body,
    out_type=jax.ShapeDtypeStruct((P_TOTAL, C), jnp.float32),
    mesh=plsc.VectorSubcoreMesh(core_axis_name="c", subcore_axis_name="s"),
    scratch_types=[
        pltpu.VMEM((CH,), jnp.float32),
        pltpu.VMEM((CH,), jnp.float32),
        pltpu.VMEM((4, CH), jnp.int32),
        pltpu.VMEM((4 * CH,), jnp.float32),
        pltpu.VMEM((4, CH, C), jnp.float32),
        pltpu.VMEM((CH, C), jnp.float32),
        pltpu.SemaphoreType.DMA,
    ],
    compiler_params=pltpu.CompilerParams(
        needs_layout_passes=False, use_tc_tiling_on_sc=False),
)


def kernel(tenInput, g):
    tab = jnp.transpose(tenInput, (0, 2, 3, 1)).reshape(P_TOTAL, C)
    gx = g[..., 0].reshape(P_TOTAL)
    gy = g[..., 1].reshape(P_TOTAL)
    out = _sampler(tab, gx, gy)
    return out.reshape(N, H, W, C).transpose(0, 3, 1, 2)
